# Initial kernel scaffold; baseline (speedup 1.0000x reference)
import jax
import jax.numpy as jnp
from jax.experimental import pallas as pl


def kernel(x, edge_index, edge_attr, pe, batch, W_node, W_pe, W_edge, WQ, WK, WV, WO, bO, WE, bE, WOe, bOe, Wn1, bn1, Wn2, bn2, We1, be1, We2, be2, Wm1, bm1, Wm2, bm2):
    raise NotImplementedError("write your pallas kernel here")


# speedup vs baseline: 39.8456x; 39.8456x over previous
import functools
import math

import jax
import jax.numpy as jnp
from jax import lax
from jax.experimental import pallas as pl
from jax.experimental.pallas import tpu as pltpu
from jax.experimental.pallas import tpu_sc as plsc

N = 10000
E = 320000
D_NODE = 128
D_EDGE = 16
D_PE = 16
HID = 128
HEADS = 8
DH = 16
NLAYER = 4
SCALE = 1.0 / math.sqrt(DH)
F32 = jnp.float32

NC = 2
NS = 16
NW = NC * NS
E_PER_W = E // NW
GB = 80
SB = 80
DEN_W = 16
N_ACC = 10240

NB = 1000
EB = 1600


@functools.lru_cache(maxsize=None)
def _sc_kernels():
    mesh = plsc.VectorSubcoreMesh(
        core_axis_name="c", subcore_axis_name="s",
        num_cores=NC, num_subcores=NS)

    @functools.partial(
        pl.kernel,
        out_type=[
            jax.ShapeDtypeStruct((E, HID), F32),
            jax.ShapeDtypeStruct((E, 2 * HID), F32),
        ],
        mesh=mesh,
        scratch_types=[
            pltpu.VMEM((GB,), jnp.int32),
            pltpu.VMEM((GB,), jnp.int32),
            pltpu.VMEM((GB,), jnp.int32),
            pltpu.VMEM((GB,), jnp.int32),
            pltpu.VMEM((GB, HID), F32),
            pltpu.VMEM((GB, HID), F32),
            pltpu.VMEM((GB, 2 * HID), F32),
            pltpu.VMEM((GB, 2 * HID), F32),
            pltpu.SemaphoreType.DMA,
            pltpu.SemaphoreType.DMA,
            pltpu.SemaphoreType.DMA,
            pltpu.SemaphoreType.DMA,
        ],
    )
    def gather_k(q_hbm, kv_hbm, dst_hbm, src_hbm, qd_hbm, kvs_hbm,
                 dsti0, srci0, dsti1, srci1, qr0, qr1, kvr0, kvr1,
                 sq0, sq1, skv0, skv1):
        c = lax.axis_index("c")
        s = lax.axis_index("s")
        base = (c * NS + s) * E_PER_W
        dsti = (dsti0, dsti1)
        srci = (srci0, srci1)
        qr = (qr0, qr1)
        kvr = (kvr0, kvr1)
        sq = (sq0, sq1)
        skv = (skv0, skv1)
        nb = E_PER_W // GB

        def do_batch(i, b):
            off = base + i * GB
            pltpu.sync_copy(dst_hbm.at[pl.ds(off, GB)], dsti[b])
            pltpu.sync_copy(src_hbm.at[pl.ds(off, GB)], srci[b])
            dq = pltpu.async_copy(q_hbm.at[dsti[b]], qr[b], sq[b])
            dkv = pltpu.async_copy(kv_hbm.at[srci[b]], kvr[b], skv[b])
            return dq, dkv

        def drain_batch(i, b, dq, dkv):
            off = base + i * GB
            dq.wait()
            pltpu.sync_copy(qr[b], qd_hbm.at[pl.ds(off, GB)])
            dkv.wait()
            pltpu.sync_copy(kvr[b], kvs_hbm.at[pl.ds(off, GB)])

        def pair_body(j, carry):
            descs = []
            for b in range(2):
                descs.append(do_batch(2 * j + b, b))
            for b in range(2):
                drain_batch(2 * j + b, b, *descs[b])
            return carry

        lax.fori_loop(0, nb // 2, pair_body, 0)
        dq, dkv = do_batch(nb - 1, 0)
        drain_batch(nb - 1, 0, dq, dkv)

    @functools.partial(
        pl.kernel,
        out_type=[
            jax.ShapeDtypeStruct((NC * N_ACC, HID), F32),
            jax.ShapeDtypeStruct((NC * N_ACC, DEN_W), F32),
        ],
        mesh=mesh,
        scratch_types=[
            pltpu.VMEM((SB,), jnp.int32),
            pltpu.VMEM((SB,), jnp.int32),
            pltpu.VMEM((SB, HID), F32),
            pltpu.VMEM((SB, HID), F32),
            pltpu.VMEM((SB, DEN_W), F32),
            pltpu.VMEM((SB, DEN_W), F32),
            pltpu.VMEM_SHARED((N_ACC, HID), F32),
            pltpu.VMEM_SHARED((N_ACC, DEN_W), F32),
            pltpu.SemaphoreType.DMA,
            pltpu.SemaphoreType.DMA,
            pltpu.SemaphoreType.DMA,
            pltpu.SemaphoreType.DMA,
        ],
        compiler_params=pltpu.CompilerParams(use_tc_tiling_on_sc=False),
    )
    def scatter_k(msg_hbm, exw_hbm, dst_hbm, znum_hbm, zden_hbm,
                  nump_hbm, denp_hbm, idx0, idx1, mr0, mr1, xr0, xr1,
                  accn, accd, sm0, sm1, sx0, sx1):
        c = lax.axis_index("c")
        s = lax.axis_index("s")
        rows = N_ACC // NS
        r0 = s * rows
        def zbody(i, carry):
            rr = r0 + i * SB
            pltpu.sync_copy(znum_hbm.at[pl.ds(rr, SB)], mr0)
            pltpu.sync_copy(mr0, accn.at[pl.ds(rr, SB)])
            pltpu.sync_copy(zden_hbm.at[pl.ds(rr, SB)], xr0)
            pltpu.sync_copy(xr0, accd.at[pl.ds(rr, SB)])
            return carry

        lax.fori_loop(0, rows // SB, zbody, 0)
        plsc.subcore_barrier()
        base = (c * NS + s) * E_PER_W
        idxv = (idx0, idx1)
        mr = (mr0, mr1)
        xr = (xr0, xr1)
        sm = (sm0, sm1)
        sx = (sx0, sx1)
        nb = E_PER_W // SB

        def fetch_batch(i, b):
            off = base + i * SB
            pltpu.sync_copy(dst_hbm.at[pl.ds(off, SB)], idxv[b])
            dm = pltpu.async_copy(msg_hbm.at[pl.ds(off, SB)], mr[b], sm[b])
            dx = pltpu.async_copy(exw_hbm.at[pl.ds(off, SB)], xr[b], sx[b])
            return dm, dx

        def add_batch(b, dm, dx):
            dm.wait()
            pltpu.sync_copy(mr[b], accn.at[idxv[b]], add=True)
            dx.wait()
            pltpu.sync_copy(xr[b], accd.at[idxv[b]], add=True)

        def pair_body(j, carry):
            descs = []
            for b in range(2):
                descs.append(fetch_batch(2 * j + b, b))
            for b in range(2):
                add_batch(b, *descs[b])
            return carry

        lax.fori_loop(0, nb // 2, pair_body, 0)
        dm, dx = fetch_batch(nb - 1, 0)
        add_batch(0, dm, dx)
        plsc.subcore_barrier()

        def obody(i, carry):
            rr = r0 + i * SB
            pltpu.sync_copy(accn.at[pl.ds(rr, SB)], mr0)
            pltpu.sync_copy(mr0, nump_hbm.at[pl.ds(c * N_ACC + rr, SB)])
            pltpu.sync_copy(accd.at[pl.ds(rr, SB)], xr0)
            pltpu.sync_copy(xr0, denp_hbm.at[pl.ds(c * N_ACC + rr, SB)])
            return carry

        lax.fori_loop(0, rows // SB, obody, 0)

    return gather_k, scatter_k


def _sc_gather(q, kv, dst, src):
    return _sc_kernels()[0](q, kv, dst, src)


def _sc_scatter(msg, exw, dst, znum, zden):
    return _sc_kernels()[1](msg, exw, dst, znum, zden)


def _ln(t):
    m = jnp.mean(t, axis=-1, keepdims=True)
    d = t - m
    v = jnp.mean(d * d, axis=-1, keepdims=True)
    return d / jnp.sqrt(v + 1e-5)


def _headmask():
    grp = lax.broadcasted_iota(jnp.int32, (HID, HEADS), 0) // DH
    col = lax.broadcasted_iota(jnp.int32, (HID, HEADS), 1)
    return (grp == col).astype(F32)


def _headmask_t():
    row = lax.broadcasted_iota(jnp.int32, (HEADS, HID), 0)
    grp = lax.broadcasted_iota(jnp.int32, (HEADS, HID), 1) // DH
    return (row == grp).astype(F32)


def _dot(a, b):
    return jnp.dot(a, b, preferred_element_type=F32)


def _node_in_body(x_ref, pe_ref, wn_ref, wp_ref, h_ref):
    h_ref[...] = _dot(x_ref[...], wn_ref[...]) + _dot(pe_ref[...], wp_ref[...])


def _edge_in_body(ea_ref, we_ref, e_ref):
    e_ref[...] = _dot(ea_ref[...], we_ref[...])


def _qkv_body(h_ref, wq_ref, wk_ref, wv_ref, q_ref, kv_ref):
    h = h_ref[...]
    q_ref[...] = _dot(h, wq_ref[...])
    kv_ref[...] = jnp.concatenate(
        [_dot(h, wk_ref[...]), _dot(h, wv_ref[...])], axis=1)


def _edge_msg_body(qd_ref, kvs_ref, msg_ref, exw_ref, p_ref):
    half = EB // 2
    for i in range(2):
        sl = pl.ds(i * half, half)
        ks = kvs_ref[sl, :HID]
        vs = kvs_ref[sl, HID:]
        p = qd_ref[sl, :] * ks
        p_ref[sl, :] = p
        ex = jnp.exp(_dot(p, _headmask()) * SCALE)
        exb = _dot(ex, _headmask_t())
        msg_ref[sl, :] = exb * vs
        exw_ref[sl, :] = jnp.concatenate(
            [ex, jnp.zeros((half, 8), F32)], axis=1)


def _edge_chain_body(e_ref, p_ref, we_ref, be_ref, woe_ref, boe_ref,
                     w1_ref, b1_ref, w2_ref, b2_ref, eout_ref):
    half = EB // 2
    for i in range(2):
        sl = pl.ds(i * half, half)
        p = p_ref[sl, :]
        e = e_ref[sl, :]
        emat = _dot(e, we_ref[...]) + be_ref[...]
        eij = emat * p * SCALE
        t = _ln(_dot(eij, woe_ref[...]) + boe_ref[...] + e)
        u = _dot(jnp.maximum(_dot(t, w1_ref[...]) + b1_ref[...], 0.0),
                 w2_ref[...]) + b2_ref[...]
        eout_ref[sl, :] = _ln(t + u)


def _node_body(hin_ref, n0_ref, n1_ref, d0_ref, d1_ref, wo_ref, bo_ref,
               w1_ref, b1_ref, w2_ref, b2_ref, h_ref):
    num = n0_ref[...] + n1_ref[...]
    den = (d0_ref[...] + d1_ref[...])[:, :HEADS]
    invb = _dot(1.0 / (den + 1e-16), _headmask_t())
    agg = num * invb
    h1 = _ln(_dot(agg, wo_ref[...]) + bo_ref[...] + hin_ref[...])
    u = _dot(jnp.maximum(_dot(h1, w1_ref[...]) + b1_ref[...], 0.0),
             w2_ref[...]) + b2_ref[...]
    h_ref[...] = _ln(h1 + u)


def _final_body(h_ref, wm1_ref, bm1_ref, wm2_ref, bm2_ref, mu_ref, acc_ref):
    i = pl.program_id(0)

    @pl.when(i == 0)
    def _():
        acc_ref[...] = jnp.zeros_like(acc_ref)

    acc_ref[...] += jnp.sum(h_ref[...], axis=0, keepdims=True)

    @pl.when(i == pl.num_programs(0) - 1)
    def _():
        g = acc_ref[...]
        t = jnp.maximum(_dot(g, wm1_ref[...]) + bm1_ref[...], 0.0)
        mu_ref[...] = _dot(t, wm2_ref[...]) + bm2_ref[...]


def _full(shape):
    return pl.BlockSpec(shape, lambda i: (0,) * len(shape))


def _rows(block_shape):
    return pl.BlockSpec(block_shape, lambda i: (i,) + (0,) * (len(block_shape) - 1))


def _node_in_call(x, pe, wn, wp):
    return pl.pallas_call(
        _node_in_body,
        grid=(N // NB,),
        in_specs=[_rows((NB, D_NODE)), _rows((NB, D_PE)),
                  _full((D_NODE, HID)), _full((D_PE, HID))],
        out_specs=_rows((NB, HID)),
        out_shape=jax.ShapeDtypeStruct((N, HID), F32),
    )(x, pe, wn, wp)


def _edge_in_call(ea, we):
    return pl.pallas_call(
        _edge_in_body,
        grid=(E // EB,),
        in_specs=[_rows((EB, D_EDGE)), _full((D_EDGE, HID))],
        out_specs=_rows((EB, HID)),
        out_shape=jax.ShapeDtypeStruct((E, HID), F32),
    )(ea, we)


def _qkv_call(h, wq, wk, wv):
    return pl.pallas_call(
        _qkv_body,
        grid=(N // NB,),
        in_specs=[_rows((NB, HID))] + [_full((HID, HID))] * 3,
        out_specs=[_rows((NB, HID)), _rows((NB, 2 * HID))],
        out_shape=[jax.ShapeDtypeStruct((N, HID), F32),
                   jax.ShapeDtypeStruct((N, 2 * HID), F32)],
    )(h, wq, wk, wv)


def _edge_msg_call(qd, kvs):
    return pl.pallas_call(
        _edge_msg_body,
        grid=(E // EB,),
        in_specs=[_rows((EB, HID)), _rows((EB, 2 * HID))],
        out_specs=[_rows((EB, HID)), _rows((EB, DEN_W)), _rows((EB, HID))],
        out_shape=[jax.ShapeDtypeStruct((E, HID), F32),
                   jax.ShapeDtypeStruct((E, DEN_W), F32),
                   jax.ShapeDtypeStruct((E, HID), F32)],
    )(qd, kvs)


def _edge_chain_call(e, p, we, be, woe, boe, w1, b1, w2, b2):
    return pl.pallas_call(
        _edge_chain_body,
        grid=(E // EB,),
        in_specs=[_rows((EB, HID)), _rows((EB, HID)),
                  _full((HID, HID)), _full((1, HID)),
                  _full((HID, HID)), _full((1, HID)),
                  _full((HID, HID)), _full((1, HID)),
                  _full((HID, HID)), _full((1, HID))],
        out_specs=_rows((EB, HID)),
        out_shape=jax.ShapeDtypeStruct((E, HID), F32),
    )(e, p, we, be, woe, boe, w1, b1, w2, b2)


def _node_call(hin, n0, n1, d0, d1, wo, bo, w1, b1, w2, b2):
    return pl.pallas_call(
        _node_body,
        grid=(N // NB,),
        in_specs=[_rows((NB, HID)), _rows((NB, HID)), _rows((NB, HID)),
                  _rows((NB, DEN_W)), _rows((NB, DEN_W)),
                  _full((HID, HID)), _full((1, HID)),
                  _full((HID, HID)), _full((1, HID)),
                  _full((HID, HID)), _full((1, HID))],
        out_specs=_rows((NB, HID)),
        out_shape=jax.ShapeDtypeStruct((N, HID), F32),
    )(hin, n0, n1, d0, d1, wo, bo, w1, b1, w2, b2)


def _final_call(h, wm1, bm1, wm2, bm2):
    return pl.pallas_call(
        _final_body,
        grid=(N // NB,),
        in_specs=[_rows((NB, HID)), _full((HID, HID)), _full((1, HID)),
                  _full((HID, 1)), _full((1, 1))],
        out_specs=_full((1, 1)),
        out_shape=jax.ShapeDtypeStruct((1, 1), F32),
        scratch_shapes=[pltpu.VMEM((1, HID), F32)],
    )(h, wm1, bm1, wm2, bm2)


def kernel(x, edge_index, edge_attr, pe, batch, W_node, W_pe, W_edge,
           WQ, WK, WV, WO, bO, WE, bE, WOe, bOe, Wn1, bn1, Wn2, bn2,
           We1, be1, We2, be2, Wm1, bm1, Wm2, bm2):
    src = edge_index[0]
    dst = edge_index[1]

    h = _node_in_call(x, pe, W_node, W_pe)
    e = _edge_in_call(edge_attr, W_edge)
    znum = jnp.zeros((N_ACC, HID), F32)
    zden = jnp.zeros((N_ACC, DEN_W), F32)

    def r2(b):
        return b.reshape(1, -1)

    for l in range(NLAYER):
        q, kv = _qkv_call(h, WQ[l], WK[l], WV[l])
        qd, kvs = _sc_gather(q, kv, dst, src)
        msg, exw, p = _edge_msg_call(qd, kvs)
        nump, denp = _sc_scatter(msg, exw, dst, znum, zden)
        e = _edge_chain_call(e, p, WE[l], r2(bE[l]), WOe[l], r2(bOe[l]),
                             We1[l], r2(be1[l]), We2[l], r2(be2[l]))
        h = _node_call(h, nump[:N_ACC], nump[N_ACC:], denp[:N_ACC],
                       denp[N_ACC:], WO[l], r2(bO[l]),
                       Wn1[l], r2(bn1[l]), Wn2[l], r2(bn2[l]))

    return _final_call(h, Wm1, r2(bm1), Wm2, r2(bm2))
